# baseline (speedup 1.0000x reference)
import functools

import jax
import jax.numpy as jnp
from jax import lax
from jax.experimental import pallas as pl
from jax.experimental.pallas import tpu as pltpu
from jax.experimental.pallas import tpu_sc as plsc

_B = 16384
_EMB = 16
_NC = 2
_NS = 16
_NW = _NC * _NS
_BPW = _B // _NW
_GROUPS = _BPW // 16


def _cf_body(u_ref, i_ref, ue_ref, ie_ref, w_ref, b_ref, out_ref,
             uidx, iidx, urows, irows, prodT, outv, wv, bv, sem):
    wid = lax.axis_index("s") * _NC + lax.axis_index("c")
    base = wid * _BPW

    pltpu.sync_copy(u_ref.at[pl.ds(base, _BPW)], uidx)
    pltpu.sync_copy(i_ref.at[pl.ds(base, _BPW)], iidx)
    cu = pltpu.async_copy(ue_ref.at[uidx], urows, sem)
    ci = pltpu.async_copy(ie_ref.at[iidx], irows, sem)
    pltpu.sync_copy(w_ref, wv)
    pltpu.sync_copy(b_ref, bv)
    cu.wait()
    ci.wait()

    lane = lax.iota(jnp.int32, 16)
    w = wv[...]
    b = bv[...]

    def g_body(g, carry):
        row = g * 16 + lane
        acc = jnp.zeros((16,), jnp.float32)
        for e in range(_EMB):
            col = jnp.full((16,), e, jnp.int32)
            uu = plsc.load_gather(urows, [row, col])
            ii = plsc.load_gather(irows, [row, col])
            acc = acc + uu * ii
        r = acc * w + b
        outv[pl.ds(g * 16, 16)] = 1.0 / (1.0 + jnp.exp(-r))
        return carry

    lax.fori_loop(0, _GROUPS, g_body, 0)
    pltpu.sync_copy(outv, out_ref.at[pl.ds(base, _BPW)])


@functools.partial(
    pl.kernel,
    out_type=jax.ShapeDtypeStruct((_B,), jnp.float32),
    mesh=plsc.VectorSubcoreMesh(core_axis_name="c", subcore_axis_name="s"),
    compiler_params=pltpu.CompilerParams(
        needs_layout_passes=False, use_tc_tiling_on_sc=False),
    scratch_types=[
        pltpu.VMEM((_BPW,), jnp.int32),
        pltpu.VMEM((_BPW,), jnp.int32),
        pltpu.VMEM((_BPW, _EMB), jnp.float32),
        pltpu.VMEM((_BPW, _EMB), jnp.float32),
        pltpu.VMEM((_BPW * _EMB,), jnp.float32),
        pltpu.VMEM((_BPW,), jnp.float32),
        pltpu.VMEM((16,), jnp.float32),
        pltpu.VMEM((16,), jnp.float32),
        pltpu.SemaphoreType.DMA,
    ],
)
def _cf_sc(u_ref, i_ref, ue_ref, ie_ref, w_ref, b_ref, out_ref, *scratch):
    _cf_body(u_ref, i_ref, ue_ref, ie_ref, w_ref, b_ref, out_ref, *scratch)


def kernel(u, i, user_emb, item_emb, W, b):
    u32 = u.astype(jnp.int32)
    i32 = i.astype(jnp.int32)
    w16 = jnp.broadcast_to(W.reshape((1,)), (16,))
    b16 = jnp.broadcast_to(b.reshape((1,)), (16,))
    out = _cf_sc(u32, i32, user_emb, item_emb, w16, b16)
    return out.reshape(_B, 1, 1)


# speedup vs baseline: 5.8200x; 5.8200x over previous
import functools

import jax
import jax.numpy as jnp
from jax import lax
from jax.experimental import pallas as pl
from jax.experimental.pallas import tpu as pltpu
from jax.experimental.pallas import tpu_sc as plsc

_B = 16384
_EMB = 16
_NC = 2
_NS = 16
_NW = _NC * _NS
_BPW = _B // _NW
_GRP = 8
_NSTG = _BPW // _GRP


def _cf_body(u_ref, i_ref, ut_ref, it_ref, w_ref, b_ref, out_ref,
             uidx, iidx, prodT, outv, wv, bv, sema, semb, *bufs):
    ubufs = bufs[:2 * _GRP]
    ibufs = bufs[2 * _GRP:]
    sems = (sema, semb)
    wid = lax.axis_index("s") * _NC + lax.axis_index("c")
    base = wid * _BPW

    pltpu.sync_copy(u_ref.at[pl.ds(base, _BPW)], uidx)
    pltpu.sync_copy(i_ref.at[pl.ds(base, _BPW)], iidx)
    pltpu.sync_copy(w_ref, wv)
    pltpu.sync_copy(b_ref, bv)

    lane = lax.iota(jnp.int32, 16)

    def fire(c, half, ph):
        uc = uidx[pl.ds(c * 16, 16)]
        ic = iidx[pl.ds(c * 16, 16)]
        for k in range(_GRP):
            ru = uc[half * _GRP + k]
            ri = ic[half * _GRP + k]
            pltpu.async_copy(
                ut_ref.at[:, pl.ds((ru >> 7) * 128, 128)],
                ubufs[ph * _GRP + k], sems[ph])
            pltpu.async_copy(
                it_ref.at[:, pl.ds((ri >> 7) * 128, 128)],
                ibufs[ph * _GRP + k], sems[ph])

    def drain(ph):
        for k in range(_GRP):
            pltpu.make_async_copy(
                ut_ref.at[:, pl.ds(0, 128)], ubufs[ph * _GRP + k], sems[ph]
            ).wait()
            pltpu.make_async_copy(
                it_ref.at[:, pl.ds(0, 128)], ibufs[ph * _GRP + k], sems[ph]
            ).wait()

    def process(c, half, ph):
        uc = uidx[pl.ds(c * 16, 16)]
        ic = iidx[pl.ds(c * 16, 16)]
        for k in range(_GRP):
            j = c * 16 + half * _GRP + k
            ru = uc[half * _GRP + k]
            ri = ic[half * _GRP + k]
            ucol = jnp.broadcast_to(ru & 127, (16,))
            icol = jnp.broadcast_to(ri & 127, (16,))
            uv = plsc.load_gather(ubufs[ph * _GRP + k], [lane, ucol])
            iv = plsc.load_gather(ibufs[ph * _GRP + k], [lane, icol])
            plsc.store_scatter(prodT, [lane * _BPW + j], uv * iv)

    fire(0, 0, 0)

    def body(t, carry):
        fire(t, 1, 1)
        drain(0)
        process(t, 0, 0)
        @pl.when(t + 1 < _NSTG // 2)
        def _():
            fire(t + 1, 0, 0)
        drain(1)
        process(t, 1, 1)
        return carry

    lax.fori_loop(0, _NSTG // 2, body, 0)

    w = wv[...]
    b = bv[...]

    def g_body(g, carry):
        sl = pl.ds(g * 16, 16)
        acc = prodT[sl]
        for e in range(1, _EMB):
            acc = acc + prodT[pl.ds(e * _BPW + g * 16, 16)]
        r = acc * w + b
        outv[sl] = 1.0 / (1.0 + jnp.exp(-r))
        return carry

    lax.fori_loop(0, _BPW // 16, g_body, 0)
    pltpu.sync_copy(outv, out_ref.at[pl.ds(base, _BPW)])


@functools.partial(
    pl.kernel,
    out_type=jax.ShapeDtypeStruct((_B,), jnp.float32),
    mesh=plsc.VectorSubcoreMesh(core_axis_name="c", subcore_axis_name="s"),
    compiler_params=pltpu.CompilerParams(needs_layout_passes=False),
    scratch_types=(
        [
            pltpu.VMEM((_BPW,), jnp.int32),
            pltpu.VMEM((_BPW,), jnp.int32),
            pltpu.VMEM((_BPW * _EMB,), jnp.float32),
            pltpu.VMEM((_BPW,), jnp.float32),
            pltpu.VMEM((16,), jnp.float32),
            pltpu.VMEM((16,), jnp.float32),
            pltpu.SemaphoreType.DMA,
            pltpu.SemaphoreType.DMA,
        ]
        + [pltpu.VMEM((16, 128), jnp.float32) for _ in range(4 * _GRP)]
    ),
)
def _cf_sc(u_ref, i_ref, ut_ref, it_ref, w_ref, b_ref, out_ref, *scratch):
    _cf_body(u_ref, i_ref, ut_ref, it_ref, w_ref, b_ref, out_ref, *scratch)


def kernel(u, i, user_emb, item_emb, W, b):
    u32 = u.astype(jnp.int32)
    i32 = i.astype(jnp.int32)
    w16 = jnp.broadcast_to(W.reshape((1,)), (16,))
    b16 = jnp.broadcast_to(b.reshape((1,)), (16,))
    out = _cf_sc(u32, i32, user_emb.T, item_emb.T, w16, b16)
    return out.reshape(_B, 1, 1)


# speedup vs baseline: 7.2328x; 1.2427x over previous
import functools

import jax
import jax.numpy as jnp
from jax import lax
from jax.experimental import pallas as pl
from jax.experimental.pallas import tpu as pltpu
from jax.experimental.pallas import tpu_sc as plsc

_B = 16384
_EMB = 16
_NC = 2
_NS = 16
_NW = _NC * _NS
_BPW = _B // _NW
_CW = 2048
_PCOLS = 7813 * 128
_SENT = jnp.int32(1 << 30)


def _gather_body(su_ref, pu_ref, si_ref, pi_ref, ut_ref, it_ref,
                 us_ref, is_ref, kidx, bidx, rows, bufa, bufb,
                 sema, semb, semw):
    bufs = (bufa, bufb)
    sems = (sema, semb)
    wid = lax.axis_index("s") * _NC + lax.axis_index("c")
    base = wid * _BPW
    lane = lax.iota(jnp.int32, 16)

    for keys_ref, perm_ref, tbl_ref, stage_ref in (
        (su_ref, pu_ref, ut_ref, us_ref),
        (si_ref, pi_ref, it_ref, is_ref),
    ):
        pltpu.sync_copy(keys_ref.at[pl.ds(base, _BPW)], kidx.at[pl.ds(0, _BPW)])
        pltpu.sync_copy(perm_ref.at[pl.ds(base, _BPW)], bidx)
        kidx[pl.ds(_BPW, 16)] = jnp.broadcast_to(_SENT, (16,))

        tile_lo = kidx[pl.ds(0, 16)][0] >> 7
        tile_hi = kidx[pl.ds(_BPW - 16, 16)][15] >> 7
        nchunks = (tile_hi - tile_lo) // 16 + 1

        def fire(c, ph):
            c0 = jnp.minimum((tile_lo + c * 16) * 128, _PCOLS - _CW)
            pltpu.async_copy(
                tbl_ref.at[:, pl.ds(c0, _CW)], bufs[ph], sems[ph])

        def drain(ph):
            pltpu.make_async_copy(
                tbl_ref.at[:, pl.ds(0, _CW)], bufs[ph], sems[ph]).wait()

        def proc(c, ph, qp):
            c0n = (tile_lo + c * 16) * 128
            c0 = jnp.minimum(c0n, _PCOLS - _CW)
            wend = c0n + _CW

            def cond(state):
                q, done = state
                fk = kidx[pl.ds(q * 16, 16)][0]
                return (done == 0) & (q < _BPW // 16) & (fk < wend)

            def wbody(state):
                q, _ = state
                vq = kidx[pl.ds(q * 16, 16)]
                vb = bidx[pl.ds(q * 16, 16)]
                for k in range(16):
                    rk = vq[k]
                    nk = rk - c0n

                    @pl.when((nk >= 0) & (nk < _CW))
                    def _():
                        slot = q * 16 + k
                        colv = jnp.broadcast_to(rk - c0, (16,))
                        v = plsc.load_gather(bufs[ph], [lane, colv])
                        rows[pl.ds(slot * 16, 16)] = v
                        pltpu.async_copy(
                            rows.at[pl.ds(slot * 16, 16)],
                            stage_ref.at[pl.ds(vb[k] * 16, 16)], semw)
                adv = (vq[15] < wend).astype(jnp.int32)
                return q + adv, 1 - adv

            q_out, _ = lax.while_loop(cond, wbody, (qp, jnp.int32(0)))
            return q_out

        fire(0, 0)

        def sbody(t, qp):
            @pl.when(2 * t + 1 < nchunks)
            def _():
                fire(2 * t + 1, 1)
            drain(0)
            qp = proc(2 * t, 0, qp)

            @pl.when(2 * t + 2 < nchunks)
            def _():
                fire(2 * t + 2, 0)

            @pl.when(2 * t + 1 < nchunks)
            def _():
                drain(1)
            qp = proc(2 * t + 1, 1, qp)
            return qp

        lax.fori_loop(0, (nchunks + 1) // 2, sbody, jnp.int32(0))

        pltpu.make_async_copy(
            stage_ref.at[pl.ds(0, _BPW * 16)], rows, semw).wait()


def _dot_body(us_ref, is_ref, w_ref, b_ref, out_ref,
              urows, irows, prodT, outv, wv, bv):
    wid = lax.axis_index("s") * _NC + lax.axis_index("c")
    base = wid * _BPW
    lane = lax.iota(jnp.int32, 16)

    pltpu.sync_copy(us_ref.at[pl.ds(base * 16, _BPW * 16)], urows)
    pltpu.sync_copy(is_ref.at[pl.ds(base * 16, _BPW * 16)], irows)
    pltpu.sync_copy(w_ref, wv)
    pltpu.sync_copy(b_ref, bv)

    def p_body(c, carry):
        for k in range(16):
            j = c * 16 + k
            uv = urows[pl.ds(j * 16, 16)]
            iv = irows[pl.ds(j * 16, 16)]
            plsc.store_scatter(prodT, [lane * _BPW + j], uv * iv)
        return carry

    lax.fori_loop(0, _BPW // 16, p_body, 0)

    w = wv[...]
    b = bv[...]

    def g_body(g, carry):
        sl = pl.ds(g * 16, 16)
        acc = prodT[sl]
        for e in range(1, _EMB):
            acc = acc + prodT[pl.ds(e * _BPW + g * 16, 16)]
        r = acc * w + b
        outv[sl] = 1.0 / (1.0 + jnp.exp(-r))
        return carry

    lax.fori_loop(0, _BPW // 16, g_body, 0)
    pltpu.sync_copy(outv, out_ref.at[pl.ds(base, _BPW)])


_MESH = plsc.VectorSubcoreMesh(core_axis_name="c", subcore_axis_name="s")
_PARAMS = pltpu.CompilerParams(needs_layout_passes=False)


@functools.partial(
    pl.kernel,
    out_type=(
        jax.ShapeDtypeStruct((_B * _EMB,), jnp.float32),
        jax.ShapeDtypeStruct((_B * _EMB,), jnp.float32),
    ),
    mesh=_MESH,
    compiler_params=_PARAMS,
    scratch_types=[
        pltpu.VMEM((_BPW + 16,), jnp.int32),
        pltpu.VMEM((_BPW,), jnp.int32),
        pltpu.VMEM((_BPW * 16,), jnp.float32),
        pltpu.VMEM((16, _CW), jnp.float32),
        pltpu.VMEM((16, _CW), jnp.float32),
        pltpu.SemaphoreType.DMA,
        pltpu.SemaphoreType.DMA,
        pltpu.SemaphoreType.DMA,
    ],
)
def _gather_sc(su, pu, si, pi, ut, it, us_out, is_out, *scratch):
    _gather_body(su, pu, si, pi, ut, it, us_out, is_out, *scratch)


@functools.partial(
    pl.kernel,
    out_type=jax.ShapeDtypeStruct((_B,), jnp.float32),
    mesh=_MESH,
    compiler_params=_PARAMS,
    scratch_types=[
        pltpu.VMEM((_BPW * 16,), jnp.float32),
        pltpu.VMEM((_BPW * 16,), jnp.float32),
        pltpu.VMEM((_BPW * _EMB,), jnp.float32),
        pltpu.VMEM((_BPW,), jnp.float32),
        pltpu.VMEM((16,), jnp.float32),
        pltpu.VMEM((16,), jnp.float32),
    ],
)
def _dot_sc(us, is_, w, b, out, *scratch):
    _dot_body(us, is_, w, b, out, *scratch)


def kernel(u, i, user_emb, item_emb, W, b):
    u32 = u.astype(jnp.int32)
    i32 = i.astype(jnp.int32)
    w16 = jnp.broadcast_to(W.reshape((1,)), (16,))
    b16 = jnp.broadcast_to(b.reshape((1,)), (16,))
    ar = jnp.arange(_B, dtype=jnp.int32)
    su, pu = jax.lax.sort_key_val(u32, ar)
    si, pi = jax.lax.sort_key_val(i32, ar)
    us, is_ = _gather_sc(su, pu, si, pi, user_emb.T, item_emb.T)
    out = _dot_sc(us, is_, w16, b16)
    return out.reshape(_B, 1, 1)
